# baseline (speedup 1.0000x reference)
import functools

import jax
import jax.numpy as jnp
from jax import lax
from jax.experimental import pallas as pl
from jax.experimental.pallas import tpu as pltpu
from jax.experimental.pallas import tpu_sc as plsc

N = 50000
E = 800000
DIN = 896
DH = 64
HALF = 32
NC = 2
NS = 16
WE = 256
NROWS = E // WE
NPAD = ((N + WE - 1) // WE) * WE
NCHUNK = NPAD // WE
KC = 1
DKC = 5
NFULLC = NROWS // KC
NDEGC = NROWS // DKC
assert NFULLC * KC == NROWS and NDEGC * DKC == NROWS

_mesh = plsc.VectorSubcoreMesh(core_axis_name="c", subcore_axis_name="s")
_sc_params = pltpu.CompilerParams(use_tc_tiling_on_sc=False)


def _split(total, parts, idx):
  base = total // parts
  rem = total - base * parts
  lo = idx * base + jnp.minimum(idx, rem)
  cnt = base + jnp.where(idx < rem, 1, 0)
  return lo, cnt


@functools.partial(
    pl.kernel,
    out_type=jax.ShapeDtypeStruct((NC, NPAD), jnp.float32),
    mesh=_mesh,
    scratch_types=[
        pltpu.VMEM((DKC, WE), jnp.int32),
        pltpu.VMEM((WE,), jnp.float32),
        pltpu.VMEM((WE,), jnp.float32),
        pltpu.VMEM_SHARED((NPAD,), jnp.float32),
        pltpu.SemaphoreType.DMA,
    ],
    compiler_params=_sc_params,
)
def _deg_call(edge3_hbm, ones_hbm, zeros_hbm, out_hbm, idx2,
              ones_v, buf, deg_sh, sem):
  c = lax.axis_index("c")
  s = lax.axis_index("s")

  pltpu.sync_copy(ones_hbm, ones_v)
  pltpu.sync_copy(zeros_hbm, buf)

  zlo, zcnt = _split(NCHUNK, NS, s)

  @pl.loop(zlo, zlo + zcnt)
  def _(k):
    pltpu.sync_copy(buf, deg_sh.at[pl.ds(pl.multiple_of(k * WE, WE), WE)])

  plsc.subcore_barrier()

  dlo, dcnt = _split(NDEGC, NC * NS, s * NC + c)

  @pl.loop(0, dcnt)
  def _(k):
    roff = (dlo + k) * DKC
    pltpu.sync_copy(edge3_hbm.at[1].at[pl.ds(roff, DKC)], idx2)
    descs = [
        pltpu.async_copy(ones_v, deg_sh.at[idx2.at[j]], sem, add=True)
        for j in range(DKC)
    ]
    for d in descs:
      d.wait()

  plsc.subcore_barrier()

  @pl.loop(zlo, zlo + zcnt)
  def _(k):
    off = pl.multiple_of(k * WE, WE)
    pltpu.sync_copy(deg_sh.at[pl.ds(off, WE)], buf)
    pltpu.sync_copy(buf, out_hbm.at[c].at[pl.ds(off, WE)])


@functools.partial(
    pl.kernel,
    out_type=jax.ShapeDtypeStruct((NC, N, HALF), jnp.float32),
    mesh=_mesh,
    scratch_types=[
        pltpu.VMEM((KC, WE), jnp.int32),
        pltpu.VMEM((KC, WE), jnp.int32),
        pltpu.VMEM((KC, WE), jnp.int32),
        pltpu.VMEM((KC, WE), jnp.int32),
        pltpu.VMEM((KC * WE, HALF), jnp.float32),
        pltpu.VMEM((KC * WE, HALF), jnp.float32),
        pltpu.VMEM_SHARED((NPAD, HALF), jnp.float32),
        pltpu.SemaphoreType.DMA,
        pltpu.SemaphoreType.DMA,
        pltpu.SemaphoreType.DMA,
        pltpu.SemaphoreType.DMA,
        pltpu.SemaphoreType.DMA,
    ],
    compiler_params=_sc_params,
)
def _agg_call(u_hbm, edge3_hbm, zeros32_hbm, out_hbm, idxrA, idxcA,
              idxrB, idxcB, rowsA, rowsB, acc_sh, semGA, semGB, semIA, semIB,
              semS):
  c = lax.axis_index("c")
  s = lax.axis_index("s")

  zv = rowsA.at[pl.ds(0, WE)]
  pltpu.sync_copy(zeros32_hbm, zv)

  zlo, zcnt = _split(NCHUNK, NS, s)

  @pl.loop(zlo, zlo + zcnt)
  def _(k):
    pltpu.sync_copy(zv, acc_sh.at[pl.ds(pl.multiple_of(k * WE, WE), WE)])

  plsc.subcore_barrier()

  clo, ccnt = _split(NFULLC, NS, s)

  def fire_idx(ch, idxr, idxc, semI):
    roff = ch * KC
    pltpu.async_copy(edge3_hbm.at[0].at[pl.ds(roff, KC)], idxr, semI)
    pltpu.async_copy(edge3_hbm.at[1].at[pl.ds(roff, KC)], idxc, semI)

  def wait_idx(idxr, idxc, semI):
    pltpu.make_async_copy(edge3_hbm.at[0].at[pl.ds(0, KC)], idxr, semI).wait()
    pltpu.make_async_copy(edge3_hbm.at[1].at[pl.ds(0, KC)], idxc, semI).wait()

  def fire_gathers(idxr, rows, semG):
    for j in range(KC):
      pltpu.async_copy(u_hbm.at[c].at[idxr.at[j]],
                       rows.at[pl.ds(j * WE, WE)], semG)

  def drain_gathers(idxr, rows, semG):
    for j in range(KC):
      pltpu.make_async_copy(u_hbm.at[c].at[idxr.at[j]],
                            rows.at[pl.ds(j * WE, WE)], semG).wait()

  def scatter_chunk(idxc, rows):
    descs = [
        pltpu.async_copy(rows.at[pl.ds(j * WE, WE)], acc_sh.at[idxc.at[j]],
                         semS, add=True)
        for j in range(KC)
    ]
    for d in descs:
      d.wait()

  A = (idxrA, idxcA, rowsA, semGA, semIA)
  B = (idxrB, idxcB, rowsB, semGB, semIB)

  fire_idx(clo, idxrA, idxcA, semIA)
  wait_idx(idxrA, idxcA, semIA)
  fire_gathers(idxrA, rowsA, semGA)

  @pl.when(ccnt > 1)
  def _():
    fire_idx(clo + 1, idxrB, idxcB, semIB)

  @pl.loop(0, ccnt)
  def _(k):
    even = (k % 2) == 0

    def halfstep(cur, nxt):
      idxr_c, idxc_c, rows_c, semG_c, semI_c = cur
      idxr_n, idxc_n, rows_n, semG_n, semI_n = nxt

      @pl.when(k + 1 < ccnt)
      def _():
        wait_idx(idxr_n, idxc_n, semI_n)
        fire_gathers(idxr_n, rows_n, semG_n)

      drain_gathers(idxr_c, rows_c, semG_c)
      scatter_chunk(idxc_c, rows_c)

      @pl.when(k + 2 < ccnt)
      def _():
        fire_idx(clo + k + 2, idxr_c, idxc_c, semI_c)

    @pl.when(even)
    def _():
      halfstep(A, B)

    @pl.when(jnp.logical_not(even))
    def _():
      halfstep(B, A)

  plsc.subcore_barrier()

  TAIL = N - (NCHUNK - 1) * WE
  wb = rowsA.at[pl.ds(0, WE)]

  @pl.loop(zlo, zlo + zcnt)
  def _(k):
    off = pl.multiple_of(k * WE, WE)

    @pl.when(k < NCHUNK - 1)
    def _():
      pltpu.sync_copy(acc_sh.at[pl.ds(off, WE)], wb)
      pltpu.sync_copy(wb, out_hbm.at[c].at[pl.ds(off, WE)])

    @pl.when(k == NCHUNK - 1)
    def _():
      pltpu.sync_copy(acc_sh.at[pl.ds(off, TAIL)], rowsA.at[pl.ds(0, TAIL)])
      pltpu.sync_copy(rowsA.at[pl.ds(0, TAIL)],
                      out_hbm.at[c].at[pl.ds(off, TAIL)])


BN = 2000
GRID = N // BN


def _norms(p0, p1):
  deg = p0 + p1 + 1.0
  dis = lax.rsqrt(deg)
  return dis, 1.0 / deg


def _tcht_body(x_ref, w1_ref, b1_ref, wg_ref, ht_ref):
  h = jnp.dot(x_ref[...], w1_ref[...], preferred_element_type=jnp.float32)
  h = h + b1_ref[...]
  ht_ref[...] = jnp.dot(h, wg_ref[...], preferred_element_type=jnp.float32)


def _tcus_body(ht_ref, bg_ref, p0_ref, p1_ref, u_ref, s_ref):
  dis, inv = _norms(p0_ref[...], p1_ref[...])
  ht = ht_ref[...]
  u = ht * dis
  u_ref[0] = u[:, :HALF]
  u_ref[1] = u[:, HALF:]
  s_ref[...] = ht * inv + bg_ref[...]


def _tcmid_body(acc_ref, sin_ref, wg_ref, bg_ref, p0_ref, p1_ref,
                u_ref, s_ref):
  dis, inv = _norms(p0_ref[...], p1_ref[...])
  accf = jnp.concatenate([acc_ref[0], acc_ref[1]], axis=1)
  h = jax.nn.relu(dis * accf + sin_ref[...])
  ht = jnp.dot(h, wg_ref[...], preferred_element_type=jnp.float32)
  u = ht * dis
  u_ref[0] = u[:, :HALF]
  u_ref[1] = u[:, HALF:]
  s_ref[...] = ht * inv + bg_ref[...]


def _tcfin_body(acc_ref, sin_ref, w2_ref, b2_ref, p0_ref, p1_ref,
                out_ref, mx_ref):
  i = pl.program_id(0)
  dis, _ = _norms(p0_ref[...], p1_ref[...])
  accf = jnp.concatenate([acc_ref[0], acc_ref[1]], axis=1)
  h = jax.nn.relu(dis * accf + sin_ref[...])
  m = jnp.max(h, axis=0, keepdims=True)

  @pl.when(i == 0)
  def _():
    mx_ref[0:1] = m

  @pl.when(i > 0)
  def _():
    mx_ref[0:1] = jnp.maximum(mx_ref[0:1], m)

  @pl.when(i == GRID - 1)
  def _():
    out_ref[...] = (
        jnp.dot(mx_ref[0:1], w2_ref[...], preferred_element_type=jnp.float32)
        + b2_ref[...])


_wspec = pl.BlockSpec((DH, DH), lambda i: (0, 0))
_bspec = pl.BlockSpec((1, DH), lambda i: (0, 0))
_pspec = pl.BlockSpec((BN, 1), lambda i: (i, 0))
_uspec = pl.BlockSpec((NC, BN, HALF), lambda i: (0, i, 0))
_sspec = pl.BlockSpec((BN, DH), lambda i: (i, 0))

_us_out = (
    jax.ShapeDtypeStruct((NC, N, HALF), jnp.float32),
    jax.ShapeDtypeStruct((N, DH), jnp.float32),
)

_tcht = pl.pallas_call(
    _tcht_body,
    grid=(GRID,),
    in_specs=[
        pl.BlockSpec((BN, DIN), lambda i: (i, 0)),
        pl.BlockSpec((DIN, DH), lambda i: (0, 0)),
        _bspec, _wspec,
    ],
    out_specs=_sspec,
    out_shape=jax.ShapeDtypeStruct((N, DH), jnp.float32),
)

_tcus = pl.pallas_call(
    _tcus_body,
    grid=(GRID,),
    in_specs=[_sspec, _bspec, _pspec, _pspec],
    out_specs=(_uspec, _sspec),
    out_shape=_us_out,
)

_tcmid = pl.pallas_call(
    _tcmid_body,
    grid=(GRID,),
    in_specs=[_uspec, _sspec, _wspec, _bspec, _pspec, _pspec],
    out_specs=(_uspec, _sspec),
    out_shape=_us_out,
)

_tcfin = pl.pallas_call(
    _tcfin_body,
    grid=(GRID,),
    in_specs=[
        _uspec, _sspec,
        pl.BlockSpec((DH, 2), lambda i: (0, 0)),
        pl.BlockSpec((1, 2), lambda i: (0, 0)),
        _pspec, _pspec,
    ],
    out_specs=pl.BlockSpec((1, 2), lambda i: (0, 0)),
    out_shape=jax.ShapeDtypeStruct((1, 2), jnp.float32),
    scratch_shapes=[pltpu.VMEM((8, DH), jnp.float32)],
)


def kernel(x, edge_index, W1, b1, Wg0, bg0, Wg1, bg1, Wg2, bg2, W2, b2):
  edge3 = edge_index.reshape(2, NROWS, WE)
  ones_row = jnp.ones((WE,), jnp.float32)
  zeros1 = jnp.zeros((WE,), jnp.float32)
  zeros32 = jnp.zeros((WE, HALF), jnp.float32)

  degp = _deg_call(edge3, ones_row, zeros1)
  p0 = degp[0, :N].reshape(N, 1)
  p1 = degp[1, :N].reshape(N, 1)

  ht0 = _tcht(x, W1, b1.reshape(1, DH), Wg0)
  u, sarr = _tcus(ht0, bg0.reshape(1, DH), p0, p1)
  acc = _agg_call(u, edge3, zeros32)
  u, sarr = _tcmid(acc, sarr, Wg1, bg1.reshape(1, DH), p0, p1)
  acc = _agg_call(u, edge3, zeros32)
  u, sarr = _tcmid(acc, sarr, Wg2, bg2.reshape(1, DH), p0, p1)
  acc = _agg_call(u, edge3, zeros32)
  return _tcfin(acc, sarr, W2, b2.reshape(1, 2), p0, p1)


# speedup vs baseline: 1.0094x; 1.0094x over previous
import functools

import jax
import jax.numpy as jnp
from jax import lax
from jax.experimental import pallas as pl
from jax.experimental.pallas import tpu as pltpu
from jax.experimental.pallas import tpu_sc as plsc

N = 50000
E = 800000
DIN = 896
DH = 64
HALF = 32
NC = 2
NS = 16
WE = 256
NROWS = E // WE
NPAD = ((N + WE - 1) // WE) * WE
NCHUNK = NPAD // WE
KC = 1
DKC = 25
NFULLC = NROWS // KC
NDEGC = NROWS // DKC
assert NFULLC * KC == NROWS and NDEGC * DKC == NROWS

_mesh = plsc.VectorSubcoreMesh(core_axis_name="c", subcore_axis_name="s")
_sc_params = pltpu.CompilerParams(use_tc_tiling_on_sc=False)


def _split(total, parts, idx):
  base = total // parts
  rem = total - base * parts
  lo = idx * base + jnp.minimum(idx, rem)
  cnt = base + jnp.where(idx < rem, 1, 0)
  return lo, cnt


@functools.partial(
    pl.kernel,
    out_type=jax.ShapeDtypeStruct((NC, NPAD), jnp.float32),
    mesh=_mesh,
    scratch_types=[
        pltpu.VMEM((DKC, WE), jnp.int32),
        pltpu.VMEM((WE,), jnp.float32),
        pltpu.VMEM((WE,), jnp.float32),
        pltpu.VMEM_SHARED((NPAD,), jnp.float32),
        pltpu.SemaphoreType.DMA,
    ],
    compiler_params=_sc_params,
)
def _deg_call(edge3_hbm, ones_hbm, zeros_hbm, out_hbm, idx2,
              ones_v, buf, deg_sh, sem):
  c = lax.axis_index("c")
  s = lax.axis_index("s")

  pltpu.sync_copy(ones_hbm, ones_v)
  pltpu.sync_copy(zeros_hbm, buf)

  zlo, zcnt = _split(NCHUNK, NS, s)

  @pl.loop(zlo, zlo + zcnt)
  def _(k):
    pltpu.sync_copy(buf, deg_sh.at[pl.ds(pl.multiple_of(k * WE, WE), WE)])

  plsc.subcore_barrier()

  dlo, dcnt = _split(NDEGC, NC * NS, s * NC + c)

  @pl.loop(0, dcnt)
  def _(k):
    roff = (dlo + k) * DKC
    pltpu.sync_copy(edge3_hbm.at[1].at[pl.ds(roff, DKC)], idx2)
    descs = [
        pltpu.async_copy(ones_v, deg_sh.at[idx2.at[j]], sem, add=True)
        for j in range(DKC)
    ]
    for d in descs:
      d.wait()

  plsc.subcore_barrier()

  @pl.loop(zlo, zlo + zcnt)
  def _(k):
    off = pl.multiple_of(k * WE, WE)
    pltpu.sync_copy(deg_sh.at[pl.ds(off, WE)], buf)
    pltpu.sync_copy(buf, out_hbm.at[c].at[pl.ds(off, WE)])


@functools.partial(
    pl.kernel,
    out_type=jax.ShapeDtypeStruct((NC, N, HALF), jnp.float32),
    mesh=_mesh,
    scratch_types=[
        pltpu.VMEM((KC, WE), jnp.int32),
        pltpu.VMEM((KC, WE), jnp.int32),
        pltpu.VMEM((KC, WE), jnp.int32),
        pltpu.VMEM((KC, WE), jnp.int32),
        pltpu.VMEM((KC * WE, HALF), jnp.float32),
        pltpu.VMEM((KC * WE, HALF), jnp.float32),
        pltpu.VMEM_SHARED((NPAD, HALF), jnp.float32),
        pltpu.SemaphoreType.DMA,
        pltpu.SemaphoreType.DMA,
        pltpu.SemaphoreType.DMA,
        pltpu.SemaphoreType.DMA,
        pltpu.SemaphoreType.DMA,
    ],
    compiler_params=_sc_params,
)
def _agg_call(u_hbm, edge3_hbm, zeros32_hbm, out_hbm, idxrA, idxcA,
              idxrB, idxcB, rowsA, rowsB, acc_sh, semGA, semGB, semIA, semIB,
              semS):
  c = lax.axis_index("c")
  s = lax.axis_index("s")

  zv = rowsA.at[pl.ds(0, WE)]
  pltpu.sync_copy(zeros32_hbm, zv)

  zlo, zcnt = _split(NCHUNK, NS, s)

  @pl.loop(zlo, zlo + zcnt)
  def _(k):
    pltpu.sync_copy(zv, acc_sh.at[pl.ds(pl.multiple_of(k * WE, WE), WE)])

  plsc.subcore_barrier()

  clo, ccnt = _split(NFULLC, NS, s)

  def fire_idx(ch, idxr, idxc, semI):
    roff = ch * KC
    pltpu.async_copy(edge3_hbm.at[0].at[pl.ds(roff, KC)], idxr, semI)
    pltpu.async_copy(edge3_hbm.at[1].at[pl.ds(roff, KC)], idxc, semI)

  def wait_idx(idxr, idxc, semI):
    pltpu.make_async_copy(edge3_hbm.at[0].at[pl.ds(0, KC)], idxr, semI).wait()
    pltpu.make_async_copy(edge3_hbm.at[1].at[pl.ds(0, KC)], idxc, semI).wait()

  def fire_gathers(idxr, rows, semG):
    for j in range(KC):
      pltpu.async_copy(u_hbm.at[c].at[idxr.at[j]],
                       rows.at[pl.ds(j * WE, WE)], semG)

  def drain_gathers(idxr, rows, semG):
    for j in range(KC):
      pltpu.make_async_copy(u_hbm.at[c].at[idxr.at[j]],
                            rows.at[pl.ds(j * WE, WE)], semG).wait()

  def scatter_chunk(idxc, rows):
    descs = [
        pltpu.async_copy(rows.at[pl.ds(j * WE, WE)], acc_sh.at[idxc.at[j]],
                         semS, add=True)
        for j in range(KC)
    ]
    for d in descs:
      d.wait()

  A = (idxrA, idxcA, rowsA, semGA, semIA)
  B = (idxrB, idxcB, rowsB, semGB, semIB)

  fire_idx(clo, idxrA, idxcA, semIA)
  wait_idx(idxrA, idxcA, semIA)
  fire_gathers(idxrA, rowsA, semGA)

  @pl.when(ccnt > 1)
  def _():
    fire_idx(clo + 1, idxrB, idxcB, semIB)

  @pl.loop(0, ccnt)
  def _(k):
    even = (k % 2) == 0

    def halfstep(cur, nxt):
      idxr_c, idxc_c, rows_c, semG_c, semI_c = cur
      idxr_n, idxc_n, rows_n, semG_n, semI_n = nxt

      @pl.when(k + 1 < ccnt)
      def _():
        wait_idx(idxr_n, idxc_n, semI_n)
        fire_gathers(idxr_n, rows_n, semG_n)

      drain_gathers(idxr_c, rows_c, semG_c)
      scatter_chunk(idxc_c, rows_c)

      @pl.when(k + 2 < ccnt)
      def _():
        fire_idx(clo + k + 2, idxr_c, idxc_c, semI_c)

    @pl.when(even)
    def _():
      halfstep(A, B)

    @pl.when(jnp.logical_not(even))
    def _():
      halfstep(B, A)

  plsc.subcore_barrier()

  TAIL = N - (NCHUNK - 1) * WE
  wb = rowsA.at[pl.ds(0, WE)]

  @pl.loop(zlo, zlo + zcnt)
  def _(k):
    off = pl.multiple_of(k * WE, WE)

    @pl.when(k < NCHUNK - 1)
    def _():
      pltpu.sync_copy(acc_sh.at[pl.ds(off, WE)], wb)
      pltpu.sync_copy(wb, out_hbm.at[c].at[pl.ds(off, WE)])

    @pl.when(k == NCHUNK - 1)
    def _():
      pltpu.sync_copy(acc_sh.at[pl.ds(off, TAIL)], rowsA.at[pl.ds(0, TAIL)])
      pltpu.sync_copy(rowsA.at[pl.ds(0, TAIL)],
                      out_hbm.at[c].at[pl.ds(off, TAIL)])


BN = 5000
GRID = N // BN


def _norms(p0, p1):
  deg = p0 + p1 + 1.0
  dis = lax.rsqrt(deg)
  return dis, 1.0 / deg


def _tcht_body(x_ref, w1_ref, b1_ref, wg_ref, ht_ref):
  h = jnp.dot(x_ref[...], w1_ref[...], preferred_element_type=jnp.float32)
  h = h + b1_ref[...]
  ht_ref[...] = jnp.dot(h, wg_ref[...], preferred_element_type=jnp.float32)


def _tcus_body(ht_ref, bg_ref, p0_ref, p1_ref, u_ref, s_ref):
  dis, inv = _norms(p0_ref[...], p1_ref[...])
  ht = ht_ref[...]
  u = ht * dis
  u_ref[0] = u[:, :HALF]
  u_ref[1] = u[:, HALF:]
  s_ref[...] = ht * inv + bg_ref[...]


def _tcmid_body(acc_ref, sin_ref, wg_ref, bg_ref, p0_ref, p1_ref,
                u_ref, s_ref):
  dis, inv = _norms(p0_ref[...], p1_ref[...])
  accf = jnp.concatenate([acc_ref[0], acc_ref[1]], axis=1)
  h = jax.nn.relu(dis * accf + sin_ref[...])
  ht = jnp.dot(h, wg_ref[...], preferred_element_type=jnp.float32)
  u = ht * dis
  u_ref[0] = u[:, :HALF]
  u_ref[1] = u[:, HALF:]
  s_ref[...] = ht * inv + bg_ref[...]


def _tcfin_body(acc_ref, sin_ref, w2_ref, b2_ref, p0_ref, p1_ref,
                out_ref, mx_ref):
  i = pl.program_id(0)
  dis, _ = _norms(p0_ref[...], p1_ref[...])
  accf = jnp.concatenate([acc_ref[0], acc_ref[1]], axis=1)
  h = jax.nn.relu(dis * accf + sin_ref[...])
  m = jnp.max(h, axis=0, keepdims=True)

  @pl.when(i == 0)
  def _():
    mx_ref[0:1] = m

  @pl.when(i > 0)
  def _():
    mx_ref[0:1] = jnp.maximum(mx_ref[0:1], m)

  @pl.when(i == GRID - 1)
  def _():
    out_ref[...] = (
        jnp.dot(mx_ref[0:1], w2_ref[...], preferred_element_type=jnp.float32)
        + b2_ref[...])


_wspec = pl.BlockSpec((DH, DH), lambda i: (0, 0))
_bspec = pl.BlockSpec((1, DH), lambda i: (0, 0))
_pspec = pl.BlockSpec((BN, 1), lambda i: (i, 0))
_uspec = pl.BlockSpec((NC, BN, HALF), lambda i: (0, i, 0))
_sspec = pl.BlockSpec((BN, DH), lambda i: (i, 0))

_us_out = (
    jax.ShapeDtypeStruct((NC, N, HALF), jnp.float32),
    jax.ShapeDtypeStruct((N, DH), jnp.float32),
)

_tcht = pl.pallas_call(
    _tcht_body,
    grid=(GRID,),
    in_specs=[
        pl.BlockSpec((BN, DIN), lambda i: (i, 0)),
        pl.BlockSpec((DIN, DH), lambda i: (0, 0)),
        _bspec, _wspec,
    ],
    out_specs=_sspec,
    out_shape=jax.ShapeDtypeStruct((N, DH), jnp.float32),
)

_tcus = pl.pallas_call(
    _tcus_body,
    grid=(GRID,),
    in_specs=[_sspec, _bspec, _pspec, _pspec],
    out_specs=(_uspec, _sspec),
    out_shape=_us_out,
)

_tcmid = pl.pallas_call(
    _tcmid_body,
    grid=(GRID,),
    in_specs=[_uspec, _sspec, _wspec, _bspec, _pspec, _pspec],
    out_specs=(_uspec, _sspec),
    out_shape=_us_out,
)

_tcfin = pl.pallas_call(
    _tcfin_body,
    grid=(GRID,),
    in_specs=[
        _uspec, _sspec,
        pl.BlockSpec((DH, 2), lambda i: (0, 0)),
        pl.BlockSpec((1, 2), lambda i: (0, 0)),
        _pspec, _pspec,
    ],
    out_specs=pl.BlockSpec((1, 2), lambda i: (0, 0)),
    out_shape=jax.ShapeDtypeStruct((1, 2), jnp.float32),
    scratch_shapes=[pltpu.VMEM((8, DH), jnp.float32)],
)


def kernel(x, edge_index, W1, b1, Wg0, bg0, Wg1, bg1, Wg2, bg2, W2, b2):
  edge3 = edge_index.reshape(2, NROWS, WE)
  ones_row = jnp.ones((WE,), jnp.float32)
  zeros1 = jnp.zeros((WE,), jnp.float32)
  zeros32 = jnp.zeros((WE, HALF), jnp.float32)

  degp = _deg_call(edge3, ones_row, zeros1)
  p0 = degp[0, :N].reshape(N, 1)
  p1 = degp[1, :N].reshape(N, 1)

  ht0 = _tcht(x, W1, b1.reshape(1, DH), Wg0)
  u, sarr = _tcus(ht0, bg0.reshape(1, DH), p0, p1)
  acc = _agg_call(u, edge3, zeros32)
  u, sarr = _tcmid(acc, sarr, Wg1, bg1.reshape(1, DH), p0, p1)
  acc = _agg_call(u, edge3, zeros32)
  u, sarr = _tcmid(acc, sarr, Wg2, bg2.reshape(1, DH), p0, p1)
  acc = _agg_call(u, edge3, zeros32)
  return _tcfin(acc, sarr, W2, b2.reshape(1, 2), p0, p1)
